# Initial kernel scaffold; baseline (speedup 1.0000x reference)
import jax
import jax.numpy as jnp
from jax.experimental import pallas as pl


def kernel(positions, atomic_numbers, emb_table, Wr0, Wp0, Ws0, Wr1, Wp1, Ws1, Wr2, Wp2, Ws2):
    raise NotImplementedError("write your pallas kernel here")


# speedup vs baseline: 36.8189x; 36.8189x over previous
import math

import jax
import jax.numpy as jnp
import numpy as np
from jax.experimental import pallas as pl
from jax.experimental.pallas import tpu as pltpu

_B = 8
_N = 96
_D = 64
_NRAD = 20
_CUT = 5.0
_ZMAX = 6
_ORBS = [0, 0, 1] * 2 + [0] * 4
_SIZE = int(sum(2 * l + 1 for l in _ORBS))

_NEED = [0, 0, 0]
for _a in _ORBS:
    for _b in _ORBS:
        for _l in range(abs(_a - _b), _a + _b + 1):
            _NEED[_l] += 1
_OFF1 = _NEED[0]
_OFF2 = _OFF1 + 3 * _NEED[1]
_FDIM = _OFF2 + 5 * _NEED[2]
_SEL9 = [0, _OFF1, _OFF1 + 1, _OFF1 + 2] + [_OFF2 + i for i in range(5)]


def _fct(n):
    return math.factorial(int(n))


def _cg_coeff(l1, m1, l2, m2, l3, m3):
    if m3 != m1 + m2 or l3 < abs(l1 - l2) or l3 > l1 + l2:
        return 0.0
    pref = math.sqrt((2 * l3 + 1) * _fct(l3 + l1 - l2) * _fct(l3 - l1 + l2)
                     * _fct(l1 + l2 - l3) / _fct(l1 + l2 + l3 + 1))
    pref *= math.sqrt(_fct(l3 + m3) * _fct(l3 - m3) * _fct(l1 - m1)
                      * _fct(l1 + m1) * _fct(l2 - m2) * _fct(l2 + m2))
    s = 0.0
    for k in range(0, l1 + l2 - l3 + 1):
        d = [k, l1 + l2 - l3 - k, l1 - m1 - k, l2 + m2 - k,
             l3 - l2 + m1 + k, l3 - l1 - m2 + k]
        if any(v < 0 for v in d):
            continue
        s += (-1.0) ** k / float(np.prod([_fct(v) for v in d]))
    return pref * s


def _cg_complex(l1, l2, l3):
    M = np.zeros((2 * l1 + 1, 2 * l2 + 1, 2 * l3 + 1))
    for m1 in range(-l1, l1 + 1):
        for m2 in range(-l2, l2 + 1):
            m3 = m1 + m2
            if abs(m3) <= l3:
                M[m1 + l1, m2 + l2, m3 + l3] = _cg_coeff(l1, m1, l2, m2, l3, m3)
    return M


def _umat(l):
    U = np.zeros((2 * l + 1, 2 * l + 1), dtype=complex)
    U[l, l] = 1.0
    for m in range(1, l + 1):
        U[l + m, l + m] = (-1) ** m / math.sqrt(2)
        U[l + m, l - m] = 1.0 / math.sqrt(2)
        U[l - m, l - m] = 1j / math.sqrt(2)
        U[l - m, l + m] = -1j * (-1) ** m / math.sqrt(2)
    return U


def _cg_real(l1, l2, l3):
    T = np.einsum('ia,jb,kc,abc->ijk', _umat(l1), _umat(l2),
                  np.conj(_umat(l3)), _cg_complex(l1, l2, l3).astype(complex))
    R = T.imag if np.linalg.norm(T.imag) > np.linalg.norm(T.real) else T.real
    return np.ascontiguousarray(R).astype(np.float32)


def _build_m9():
    cg = {}
    for l1 in range(2):
        for l2 in range(2):
            for l3 in range(abs(l1 - l2), l1 + l2 + 1):
                cg[(l1, l2, l3)] = _cg_real(l1, l2, l3)
    mh = np.zeros((9, _SIZE, _SIZE), dtype=np.float64)
    ci = 0
    for di in _ORBS:
        cj = 0
        for dj in _ORBS:
            for l3 in range(abs(di - dj), di + dj + 1):
                base = {0: 0, 1: 1, 2: 4}[l3]
                C = cg[(di, dj, l3)]
                for kk in range(2 * l3 + 1):
                    mh[base + kk, ci:ci + 2 * di + 1, cj:cj + 2 * dj + 1] += C[:, :, kk]
            cj += 2 * dj + 1
        ci += 2 * di + 1
    msym = mh + np.transpose(mh, (0, 2, 1))
    return msym.reshape(9, _SIZE * _SIZE).astype(np.float32)


_M9 = _build_m9()


def _so3_body(pr_ref, pc_ref, an_ref, emb_ref, wr_ref,
              wp0_ref, ws0_ref, wp1_ref, ws1_ref, wp2_ref, ws2_ref,
              m9_ref, out_ref):
    f32 = jnp.float32
    pr = pr_ref[0]
    pc = pc_ref[0]

    dx = pc[:, 0:1] - pr[0:1, :]
    dy = pc[:, 1:2] - pr[1:2, :]
    dz = pc[:, 2:3] - pr[2:3, :]
    r = jnp.sqrt(dx * dx + dy * dy + dz * dz + 1e-12)
    inv_r = 1.0 / r
    ux = dx * inv_r
    uy = dy * inv_r
    uz = dz * inv_r

    row_i = jax.lax.broadcasted_iota(jnp.int32, (_N, _N), 0)
    col_i = jax.lax.broadcasted_iota(jnp.int32, (_N, _N), 1)
    mask = (row_i > col_i).astype(f32)
    s3 = math.sqrt(3.0)
    s15 = math.sqrt(15.0)
    s5 = math.sqrt(5.0)
    sh = [
        mask,
        s3 * uy * mask,
        s3 * uz * mask,
        s3 * ux * mask,
        s15 * ux * uy * mask,
        s15 * uy * uz * mask,
        (0.5 * s5) * (3.0 * uz * uz - 1.0) * mask,
        s15 * ux * uz * mask,
        (0.5 * s15) * (ux * ux - uy * uy) * mask,
    ]

    fcut = 0.5 * (jnp.cos(jnp.pi * jnp.clip(r * (1.0 / _CUT), 0.0, 1.0)) + 1.0)
    inv_w = _NRAD / _CUT
    g = [jnp.zeros((_N, _N), f32) for _ in range(27)]
    for t in range(_NRAD):
        c_t = _CUT * t / (_NRAD - 1)
        a = (r - c_t) * inv_w
        e = jnp.exp(-(a * a)) * fcut
        for j in range(27):
            g[j] = g[j] + e * wr_ref[t:t + 1, j:j + 1]

    an = an_ref[0]
    zi = jax.lax.broadcasted_iota(jnp.int32, (_N, _ZMAX + 1), 1)
    oh = (an == zi).astype(f32)
    x = jnp.dot(oh, emb_ref[:, :], preferred_element_type=f32)

    inv_sqrt_n = 1.0 / math.sqrt(_N)

    for l, (wp, ws) in enumerate(((wp0_ref, ws0_ref), (wp1_ref, ws1_ref))):
        agg = jnp.zeros((_N, _D), f32)
        for k in range(9):
            A = sh[k] * g[9 * l + k]
            y = jnp.dot(x, wp[_D * k:_D * (k + 1), :], preferred_element_type=f32)
            agg = agg + jnp.dot(A, y, preferred_element_type=f32)
        x = agg * inv_sqrt_n + jnp.dot(x, ws[:, :], preferred_element_type=f32)

    vrows = []
    for k in range(9):
        A = sh[k] * g[18 + k]
        vrows.append(jnp.sum(A, axis=0, keepdims=True))
    v = jnp.concatenate(vrows, axis=0)
    t9 = jnp.dot(v, x, preferred_element_type=f32)
    sum_agg = jnp.zeros((1, 9), f32)
    for k in range(9):
        sum_agg = sum_agg + jnp.dot(t9[k:k + 1, :], wp2_ref[_D * k:_D * (k + 1), :],
                                    preferred_element_type=f32)
    mean_x = jnp.mean(x, axis=0, keepdims=True)
    xbar9 = sum_agg * (1.0 / (_N * math.sqrt(_N))) \
        + jnp.dot(mean_x, ws2_ref[:, :], preferred_element_type=f32)
    out_ref[0] = jnp.dot(xbar9, m9_ref[:, :], preferred_element_type=f32)


def kernel(positions, atomic_numbers, emb_table, Wr0, Wp0, Ws0,
           Wr1, Wp1, Ws1, Wr2, Wp2, Ws2):
    f32 = jnp.float32
    pos_row = jnp.swapaxes(positions, 1, 2)
    pos_col = positions
    an = atomic_numbers.astype(jnp.int32)[..., None]
    wr_all = jnp.concatenate([Wr0, Wr1, Wr2], axis=1)
    sel = jnp.asarray(_SEL9, dtype=jnp.int32)
    wp2s = Wp2[:, sel]
    ws2s = Ws2[:, sel]
    m9 = jnp.asarray(_M9, dtype=f32)

    bcast = lambda *shape: pl.BlockSpec(shape, lambda b: (0,) * len(shape))
    out = pl.pallas_call(
        _so3_body,
        grid=(_B,),
        in_specs=[
            pl.BlockSpec((1, 3, _N), lambda b: (b, 0, 0)),
            pl.BlockSpec((1, _N, 3), lambda b: (b, 0, 0)),
            pl.BlockSpec((1, _N, 1), lambda b: (b, 0, 0)),
            bcast(_ZMAX + 1, _D),
            bcast(_NRAD, 27),
            bcast(9 * _D, _D),
            bcast(_D, _D),
            bcast(9 * _D, _D),
            bcast(_D, _D),
            bcast(9 * _D, 9),
            bcast(_D, 9),
            bcast(9, _SIZE * _SIZE),
        ],
        out_specs=pl.BlockSpec((1, 1, _SIZE * _SIZE), lambda b: (b, 0, 0)),
        out_shape=jax.ShapeDtypeStruct((_B, 1, _SIZE * _SIZE), f32),
        compiler_params=pltpu.CompilerParams(
            dimension_semantics=("parallel",)),
    )(pos_row, pos_col, an, emb_table, wr_all,
      Wp0, Ws0, Wp1, Ws1, wp2s, ws2s, m9)
    return out.reshape(_B, _SIZE * _SIZE)


# speedup vs baseline: 39.9134x; 1.0840x over previous
import math

import jax
import jax.numpy as jnp
import numpy as np
from jax.experimental import pallas as pl
from jax.experimental.pallas import tpu as pltpu

_B = 8
_N = 96
_D = 64
_NRAD = 20
_CUT = 5.0
_ZMAX = 6
_ORBS = [0, 0, 1] * 2 + [0] * 4
_SIZE = int(sum(2 * l + 1 for l in _ORBS))

_NEED = [0, 0, 0]
for _a in _ORBS:
    for _b in _ORBS:
        for _l in range(abs(_a - _b), _a + _b + 1):
            _NEED[_l] += 1
_OFF1 = _NEED[0]
_OFF2 = _OFF1 + 3 * _NEED[1]
_FDIM = _OFF2 + 5 * _NEED[2]
_SEL9 = [0, _OFF1, _OFF1 + 1, _OFF1 + 2] + [_OFF2 + i for i in range(5)]


def _fct(n):
    return math.factorial(int(n))


def _cg_coeff(l1, m1, l2, m2, l3, m3):
    if m3 != m1 + m2 or l3 < abs(l1 - l2) or l3 > l1 + l2:
        return 0.0
    pref = math.sqrt((2 * l3 + 1) * _fct(l3 + l1 - l2) * _fct(l3 - l1 + l2)
                     * _fct(l1 + l2 - l3) / _fct(l1 + l2 + l3 + 1))
    pref *= math.sqrt(_fct(l3 + m3) * _fct(l3 - m3) * _fct(l1 - m1)
                      * _fct(l1 + m1) * _fct(l2 - m2) * _fct(l2 + m2))
    s = 0.0
    for k in range(0, l1 + l2 - l3 + 1):
        d = [k, l1 + l2 - l3 - k, l1 - m1 - k, l2 + m2 - k,
             l3 - l2 + m1 + k, l3 - l1 - m2 + k]
        if any(v < 0 for v in d):
            continue
        s += (-1.0) ** k / float(np.prod([_fct(v) for v in d]))
    return pref * s


def _cg_complex(l1, l2, l3):
    M = np.zeros((2 * l1 + 1, 2 * l2 + 1, 2 * l3 + 1))
    for m1 in range(-l1, l1 + 1):
        for m2 in range(-l2, l2 + 1):
            m3 = m1 + m2
            if abs(m3) <= l3:
                M[m1 + l1, m2 + l2, m3 + l3] = _cg_coeff(l1, m1, l2, m2, l3, m3)
    return M


def _umat(l):
    U = np.zeros((2 * l + 1, 2 * l + 1), dtype=complex)
    U[l, l] = 1.0
    for m in range(1, l + 1):
        U[l + m, l + m] = (-1) ** m / math.sqrt(2)
        U[l + m, l - m] = 1.0 / math.sqrt(2)
        U[l - m, l - m] = 1j / math.sqrt(2)
        U[l - m, l + m] = -1j * (-1) ** m / math.sqrt(2)
    return U


def _cg_real(l1, l2, l3):
    T = np.einsum('ia,jb,kc,abc->ijk', _umat(l1), _umat(l2),
                  np.conj(_umat(l3)), _cg_complex(l1, l2, l3).astype(complex))
    R = T.imag if np.linalg.norm(T.imag) > np.linalg.norm(T.real) else T.real
    return np.ascontiguousarray(R).astype(np.float32)


def _build_readout():
    cg = {}
    for l1 in range(2):
        for l2 in range(2):
            for l3 in range(abs(l1 - l2), l1 + l2 + 1):
                cg[(l1, l2, l3)] = _cg_real(l1, l2, l3)
    mh = np.zeros((9, _SIZE, _SIZE), dtype=np.float64)
    ci = 0
    for di in _ORBS:
        cj = 0
        for dj in _ORBS:
            for l3 in range(abs(di - dj), di + dj + 1):
                base = {0: 0, 1: 1, 2: 4}[l3]
                C = cg[(di, dj, l3)]
                for kk in range(2 * l3 + 1):
                    mh[base + kk, ci:ci + 2 * di + 1, cj:cj + 2 * dj + 1] += C[:, :, kk]
            cj += 2 * dj + 1
        ci += 2 * di + 1
    msym = mh + np.transpose(mh, (0, 2, 1))
    m9 = msym.reshape(9, _SIZE * _SIZE)
    full = np.zeros((_FDIM, _SIZE * _SIZE), dtype=np.float64)
    for i, ch in enumerate(_SEL9):
        full[ch, :] = m9[i]
    return full.astype(np.float32)


_M196 = _build_readout()


def _so3_body(pc_ref, an_ref, emb_ref, wr0_ref, wp0_ref, ws0_ref,
              wr1_ref, wp1_ref, ws1_ref, wr2_ref, wp2_ref, ws2_ref,
              m_ref, out_ref):
    f32 = jnp.float32
    pc = pc_ref[0]
    pr = jnp.swapaxes(pc, 0, 1)

    dx = pc[:, 0:1] - pr[0:1, :]
    dy = pc[:, 1:2] - pr[1:2, :]
    dz = pc[:, 2:3] - pr[2:3, :]
    r2 = dx * dx + dy * dy + dz * dz + 1e-12
    inv_r = jax.lax.rsqrt(r2)
    r = r2 * inv_r
    ux = dx * inv_r
    uy = dy * inv_r
    uz = dz * inv_r

    row_i = jax.lax.broadcasted_iota(jnp.int32, (_N, _N), 0)
    col_i = jax.lax.broadcasted_iota(jnp.int32, (_N, _N), 1)
    mask = (row_i > col_i).astype(f32)
    s3 = math.sqrt(3.0)
    s15 = math.sqrt(15.0)
    s5 = math.sqrt(5.0)
    sh = [
        mask,
        s3 * uy * mask,
        s3 * uz * mask,
        s3 * ux * mask,
        s15 * ux * uy * mask,
        s15 * uy * uz * mask,
        (0.5 * s5) * (3.0 * uz * uz - 1.0) * mask,
        s15 * ux * uz * mask,
        (0.5 * s15) * (ux * ux - uy * uy) * mask,
    ]

    fcut = 0.5 * (jnp.cos(jnp.pi * jnp.clip(r * (1.0 / _CUT), 0.0, 1.0)) + 1.0)
    inv_w = _NRAD / _CUT

    def gates(wr_ref):
        g = [jnp.zeros((_N, _N), f32) for _ in range(9)]
        for t in range(_NRAD):
            c_t = _CUT * t / (_NRAD - 1)
            a = (r - c_t) * inv_w
            e = jnp.exp(-(a * a)) * fcut
            for k in range(9):
                g[k] = g[k] + e * wr_ref[t:t + 1, k:k + 1]
        return g

    an = jnp.swapaxes(an_ref[0], 0, 1)
    zi = jax.lax.broadcasted_iota(jnp.int32, (_N, _ZMAX + 1), 1)
    oh = (an == zi).astype(f32)
    x = jnp.dot(oh, emb_ref[:, :], preferred_element_type=f32)

    inv_sqrt_n = 1.0 / math.sqrt(_N)

    for wr, wp, ws in ((wr0_ref, wp0_ref, ws0_ref), (wr1_ref, wp1_ref, ws1_ref)):
        g = gates(wr)
        agg = jnp.zeros((_N, _D), f32)
        for k in range(9):
            A = sh[k] * g[k]
            y = jnp.dot(x, wp[_D * k:_D * (k + 1), :], preferred_element_type=f32)
            agg = agg + jnp.dot(A, y, preferred_element_type=f32)
        x = agg * inv_sqrt_n + jnp.dot(x, ws[:, :], preferred_element_type=f32)

    g = gates(wr2_ref)
    vrows = []
    for k in range(9):
        A = sh[k] * g[k]
        vrows.append(jnp.sum(A, axis=0, keepdims=True))
    v = jnp.concatenate(vrows, axis=0)
    t9 = jnp.dot(v, x, preferred_element_type=f32)
    sum_agg = jnp.zeros((1, _FDIM), f32)
    for k in range(9):
        sum_agg = sum_agg + jnp.dot(t9[k:k + 1, :], wp2_ref[_D * k:_D * (k + 1), :],
                                    preferred_element_type=f32)
    mean_x = jnp.mean(x, axis=0, keepdims=True)
    xbar = sum_agg * (1.0 / (_N * math.sqrt(_N))) \
        + jnp.dot(mean_x, ws2_ref[:, :], preferred_element_type=f32)
    out_ref[0] = jnp.dot(xbar, m_ref[:, :], preferred_element_type=f32)


def kernel(positions, atomic_numbers, emb_table, Wr0, Wp0, Ws0,
           Wr1, Wp1, Ws1, Wr2, Wp2, Ws2):
    f32 = jnp.float32
    an = atomic_numbers.astype(jnp.int32).reshape(_B, 1, _N)
    m = jnp.asarray(_M196, dtype=f32)

    bcast = lambda *shape: pl.BlockSpec(shape, lambda b: (0,) * len(shape))
    out = pl.pallas_call(
        _so3_body,
        grid=(_B,),
        in_specs=[
            pl.BlockSpec((1, _N, 3), lambda b: (b, 0, 0)),
            pl.BlockSpec((1, 1, _N), lambda b: (b, 0, 0)),
            bcast(_ZMAX + 1, _D),
            bcast(_NRAD, 9),
            bcast(9 * _D, _D),
            bcast(_D, _D),
            bcast(_NRAD, 9),
            bcast(9 * _D, _D),
            bcast(_D, _D),
            bcast(_NRAD, 9),
            bcast(9 * _D, _FDIM),
            bcast(_D, _FDIM),
            bcast(_FDIM, _SIZE * _SIZE),
        ],
        out_specs=pl.BlockSpec((1, 1, _SIZE * _SIZE), lambda b: (b, 0, 0)),
        out_shape=jax.ShapeDtypeStruct((_B, 1, _SIZE * _SIZE), f32),
        compiler_params=pltpu.CompilerParams(
            dimension_semantics=("parallel",)),
    )(positions, an, emb_table, Wr0, Wp0, Ws0, Wr1, Wp1, Ws1, Wr2, Wp2, Ws2, m)
    return out.reshape(_B, _SIZE * _SIZE)


# speedup vs baseline: 43.0226x; 1.0779x over previous
import math

import jax
import jax.numpy as jnp
import numpy as np
from jax.experimental import pallas as pl
from jax.experimental.pallas import tpu as pltpu

_B = 8
_N = 96
_D = 64
_NRAD = 20
_CUT = 5.0
_ZMAX = 6
_ORBS = [0, 0, 1] * 2 + [0] * 4
_SIZE = int(sum(2 * l + 1 for l in _ORBS))

_NEED = [0, 0, 0]
for _a in _ORBS:
    for _b in _ORBS:
        for _l in range(abs(_a - _b), _a + _b + 1):
            _NEED[_l] += 1
_OFF1 = _NEED[0]
_OFF2 = _OFF1 + 3 * _NEED[1]
_FDIM = _OFF2 + 5 * _NEED[2]
_SEL9 = [0, _OFF1, _OFF1 + 1, _OFF1 + 2] + [_OFF2 + i for i in range(5)]


def _fct(n):
    return math.factorial(int(n))


def _cg_coeff(l1, m1, l2, m2, l3, m3):
    if m3 != m1 + m2 or l3 < abs(l1 - l2) or l3 > l1 + l2:
        return 0.0
    pref = math.sqrt((2 * l3 + 1) * _fct(l3 + l1 - l2) * _fct(l3 - l1 + l2)
                     * _fct(l1 + l2 - l3) / _fct(l1 + l2 + l3 + 1))
    pref *= math.sqrt(_fct(l3 + m3) * _fct(l3 - m3) * _fct(l1 - m1)
                      * _fct(l1 + m1) * _fct(l2 - m2) * _fct(l2 + m2))
    s = 0.0
    for k in range(0, l1 + l2 - l3 + 1):
        d = [k, l1 + l2 - l3 - k, l1 - m1 - k, l2 + m2 - k,
             l3 - l2 + m1 + k, l3 - l1 - m2 + k]
        if any(v < 0 for v in d):
            continue
        s += (-1.0) ** k / float(np.prod([_fct(v) for v in d]))
    return pref * s


def _cg_complex(l1, l2, l3):
    M = np.zeros((2 * l1 + 1, 2 * l2 + 1, 2 * l3 + 1))
    for m1 in range(-l1, l1 + 1):
        for m2 in range(-l2, l2 + 1):
            m3 = m1 + m2
            if abs(m3) <= l3:
                M[m1 + l1, m2 + l2, m3 + l3] = _cg_coeff(l1, m1, l2, m2, l3, m3)
    return M


def _umat(l):
    U = np.zeros((2 * l + 1, 2 * l + 1), dtype=complex)
    U[l, l] = 1.0
    for m in range(1, l + 1):
        U[l + m, l + m] = (-1) ** m / math.sqrt(2)
        U[l + m, l - m] = 1.0 / math.sqrt(2)
        U[l - m, l - m] = 1j / math.sqrt(2)
        U[l - m, l + m] = -1j * (-1) ** m / math.sqrt(2)
    return U


def _cg_real(l1, l2, l3):
    T = np.einsum('ia,jb,kc,abc->ijk', _umat(l1), _umat(l2),
                  np.conj(_umat(l3)), _cg_complex(l1, l2, l3).astype(complex))
    R = T.imag if np.linalg.norm(T.imag) > np.linalg.norm(T.real) else T.real
    return np.ascontiguousarray(R).astype(np.float32)


def _build_readout():
    cg = {}
    for l1 in range(2):
        for l2 in range(2):
            for l3 in range(abs(l1 - l2), l1 + l2 + 1):
                cg[(l1, l2, l3)] = _cg_real(l1, l2, l3)
    mh = np.zeros((9, _SIZE, _SIZE), dtype=np.float64)
    ci = 0
    for di in _ORBS:
        cj = 0
        for dj in _ORBS:
            for l3 in range(abs(di - dj), di + dj + 1):
                base = {0: 0, 1: 1, 2: 4}[l3]
                C = cg[(di, dj, l3)]
                for kk in range(2 * l3 + 1):
                    mh[base + kk, ci:ci + 2 * di + 1, cj:cj + 2 * dj + 1] += C[:, :, kk]
            cj += 2 * dj + 1
        ci += 2 * di + 1
    msym = mh + np.transpose(mh, (0, 2, 1))
    m9 = msym.reshape(9, _SIZE * _SIZE)
    full = np.zeros((_FDIM, _SIZE * _SIZE), dtype=np.float64)
    for i, ch in enumerate(_SEL9):
        full[ch, :] = m9[i]
    return full.astype(np.float32)


_M196 = _build_readout()

_BBLK = 2


def _so3_body(pc_ref, an_ref, emb_ref, wr0_ref, wp0_ref, ws0_ref,
              wr1_ref, wp1_ref, ws1_ref, wr2_ref, wp2_ref, ws2_ref,
              m_ref, out_ref):
    for i in range(_BBLK):
        _so3_one(i, pc_ref, an_ref, emb_ref, wr0_ref, wp0_ref, ws0_ref,
                 wr1_ref, wp1_ref, ws1_ref, wr2_ref, wp2_ref, ws2_ref,
                 m_ref, out_ref)


def _so3_one(i, pc_ref, an_ref, emb_ref, wr0_ref, wp0_ref, ws0_ref,
             wr1_ref, wp1_ref, ws1_ref, wr2_ref, wp2_ref, ws2_ref,
             m_ref, out_ref):
    f32 = jnp.float32
    pc = pc_ref[i]
    pr = jnp.swapaxes(pc, 0, 1)

    dx = pc[:, 0:1] - pr[0:1, :]
    dy = pc[:, 1:2] - pr[1:2, :]
    dz = pc[:, 2:3] - pr[2:3, :]
    r2 = dx * dx + dy * dy + dz * dz + 1e-12
    inv_r = jax.lax.rsqrt(r2)
    r = r2 * inv_r
    ux = dx * inv_r
    uy = dy * inv_r
    uz = dz * inv_r

    s3 = math.sqrt(3.0)
    s15 = math.sqrt(15.0)
    s5 = math.sqrt(5.0)
    sh = [
        None,
        s3 * uy,
        s3 * uz,
        s3 * ux,
        s15 * ux * uy,
        s15 * uy * uz,
        (0.5 * s5) * (3.0 * uz * uz - 1.0),
        s15 * ux * uz,
        (0.5 * s15) * (ux * ux - uy * uy),
    ]

    row_i = jax.lax.broadcasted_iota(jnp.int32, (_N, _N), 0)
    col_i = jax.lax.broadcasted_iota(jnp.int32, (_N, _N), 1)
    mask = (row_i > col_i).astype(f32)
    fcut = 0.5 * (jnp.cos(jnp.pi * jnp.clip(r * (1.0 / _CUT), 0.0, 1.0)) + 1.0) * mask
    inv_w = _NRAD / _CUT

    def gates(wr_ref):
        g = [jnp.zeros((_N, _N), f32) for _ in range(9)]
        for t in range(_NRAD):
            c_t = _CUT * t / (_NRAD - 1)
            a = (r - c_t) * inv_w
            e = jnp.exp(-(a * a)) * fcut
            for k in range(9):
                g[k] = g[k] + e * wr_ref[t:t + 1, k:k + 1]
        return g

    an = jnp.swapaxes(an_ref[i], 0, 1)
    zi = jax.lax.broadcasted_iota(jnp.int32, (_N, _ZMAX + 1), 1)
    oh = (an == zi).astype(f32)
    x = jnp.dot(oh, emb_ref[:, :], preferred_element_type=f32)

    inv_sqrt_n = 1.0 / math.sqrt(_N)

    for wr, wp, ws in ((wr0_ref, wp0_ref, ws0_ref), (wr1_ref, wp1_ref, ws1_ref)):
        g = gates(wr)
        agg = jnp.zeros((_N, _D), f32)
        for k in range(9):
            A = g[k] if k == 0 else sh[k] * g[k]
            y = jnp.dot(x, wp[_D * k:_D * (k + 1), :], preferred_element_type=f32)
            agg = agg + jnp.dot(A, y, preferred_element_type=f32)
        x = agg * inv_sqrt_n + jnp.dot(x, ws[:, :], preferred_element_type=f32)

    g = gates(wr2_ref)
    vrows = []
    for k in range(9):
        A = g[k] if k == 0 else sh[k] * g[k]
        vrows.append(jnp.sum(A, axis=0, keepdims=True))
    v = jnp.concatenate(vrows, axis=0)
    t9 = jnp.dot(v, x, preferred_element_type=f32)
    sum_agg = jnp.zeros((1, _FDIM), f32)
    for k in range(9):
        sum_agg = sum_agg + jnp.dot(t9[k:k + 1, :], wp2_ref[_D * k:_D * (k + 1), :],
                                    preferred_element_type=f32)
    mean_x = jnp.mean(x, axis=0, keepdims=True)
    xbar = sum_agg * (1.0 / (_N * math.sqrt(_N))) \
        + jnp.dot(mean_x, ws2_ref[:, :], preferred_element_type=f32)
    out_ref[i] = jnp.dot(xbar, m_ref[:, :], preferred_element_type=f32)


def kernel(positions, atomic_numbers, emb_table, Wr0, Wp0, Ws0,
           Wr1, Wp1, Ws1, Wr2, Wp2, Ws2):
    f32 = jnp.float32
    an = atomic_numbers.astype(jnp.int32).reshape(_B, 1, _N)
    m = jnp.asarray(_M196, dtype=f32)

    bcast = lambda *shape: pl.BlockSpec(shape, lambda b: (0,) * len(shape))
    out = pl.pallas_call(
        _so3_body,
        grid=(_B // _BBLK,),
        in_specs=[
            pl.BlockSpec((_BBLK, _N, 3), lambda b: (b, 0, 0)),
            pl.BlockSpec((_BBLK, 1, _N), lambda b: (b, 0, 0)),
            bcast(_ZMAX + 1, _D),
            bcast(_NRAD, 9),
            bcast(9 * _D, _D),
            bcast(_D, _D),
            bcast(_NRAD, 9),
            bcast(9 * _D, _D),
            bcast(_D, _D),
            bcast(_NRAD, 9),
            bcast(9 * _D, _FDIM),
            bcast(_D, _FDIM),
            bcast(_FDIM, _SIZE * _SIZE),
        ],
        out_specs=pl.BlockSpec((_BBLK, 1, _SIZE * _SIZE), lambda b: (b, 0, 0)),
        out_shape=jax.ShapeDtypeStruct((_B, 1, _SIZE * _SIZE), f32),
        compiler_params=pltpu.CompilerParams(
            dimension_semantics=("parallel",)),
    )(positions, an, emb_table, Wr0, Wp0, Ws0, Wr1, Wp1, Ws1, Wr2, Wp2, Ws2, m)
    return out.reshape(_B, _SIZE * _SIZE)
